# Initial kernel scaffold; baseline (speedup 1.0000x reference)
import jax
import jax.numpy as jnp
from jax.experimental import pallas as pl


def kernel(pred_logits, pred_boxes, orig_sizes):
    raise NotImplementedError("write your pallas kernel here")


# speedup vs baseline: 6.7401x; 6.7401x over previous
import functools

import jax
import jax.numpy as jnp
from jax import lax
from jax.experimental import pallas as pl
from jax.experimental.pallas import tpu as pltpu
from jax.experimental.pallas import tpu_sc as plsc

B = 16
NQ = 20000
NC_CLS = 80
FLAT = NQ * NC_CLS
TOPK = 300

NCORE = 2
NSUB = 16
NW = NCORE * NSUB
SLICE = FLAT // NW
CHUNK = 10_000
NCHUNK = SLICE // CHUNK
NB = 4096
CAPW = 256
CAP = NW * CAPW
WPAD = 320
ORD = 384
OUTW = 304

_MESH = plsc.VectorSubcoreMesh(core_axis_name="c", subcore_axis_name="s")
_CPARAMS = pltpu.CompilerParams(needs_layout_passes=False)

_I32 = jnp.int32
_F32 = jnp.float32


def _wid():
    return lax.axis_index("s") * NCORE + lax.axis_index("c")


def _iota16():
    return lax.iota(_I32, 16)


def _ones16():
    return jnp.ones((16,), _I32)


def _smk(bits):
    return bits ^ (lax.shift_right_arithmetic(bits, 31) & jnp.int32(0x7FFFFFFF))


def _k1_body(flat_hbm, hist_hbm, buf, hist):
    wid = _wid()
    base_col = wid * SLICE

    def batch_loop(b, _):
        def zero_loop(j, _):
            hist[pl.ds(j * 16, 16)] = jnp.zeros((16,), _I32)
            return 0

        lax.fori_loop(0, NB // 16, zero_loop, 0)

        def chunk_loop(c, _):
            pltpu.sync_copy(
                flat_hbm.at[pl.ds(b * FLAT + base_col + c * CHUNK, CHUNK)], buf
            )

            def vec_loop(i, _):
                v = buf[pl.ds(i * 16, 16)]
                bu = lax.bitcast_convert_type(v, jnp.uint32)
                bucket = (bu >> jnp.uint32(20)).astype(_I32)
                plsc.addupdate_scatter(hist, [bucket], _ones16())
                return 0

            lax.fori_loop(0, CHUNK // 16, vec_loop, 0)
            return 0

        lax.fori_loop(0, NCHUNK, chunk_loop, 0)
        pltpu.sync_copy(hist, hist_hbm.at[pl.ds((wid * B + b) * NB, NB)])
        return 0

    lax.fori_loop(0, B, batch_loop, 0)


def _k2_body(hist_hbm, ethr_hbm, hbuf, acc, ebuf):
    wid = _wid()

    @pl.when(wid < B)
    def _():
        b = wid

        def zero_loop(j, _):
            acc[pl.ds(j * 16, 16)] = jnp.zeros((16,), _I32)
            return 0

        lax.fori_loop(0, NB // 16, zero_loop, 0)

        def accum_loop(w, _):
            pltpu.sync_copy(hist_hbm.at[pl.ds((w * B + b) * NB, NB)], hbuf)

            def add_loop(j, _):
                acc[pl.ds(j * 16, 16)] += hbuf[pl.ds(j * 16, 16)]
                return 0

            lax.fori_loop(0, NB // 16, add_loop, 0)
            return 0

        lax.fori_loop(0, NW, accum_loop, 0)

        need = jnp.int32(TOPK)

        def scan_loop(j, carry):
            cnt_before, efound = carry
            is_pos = j < 128
            lo_pos = jnp.maximum(2032 - 16 * j, 0)
            lo_neg = jnp.clip(2048 + 16 * (j - 128), 2048, NB - 16)
            vpos = lax.rev(acc[pl.ds(lo_pos, 16)], (0,))
            vneg = acc[pl.ds(lo_neg, 16)]
            v = jnp.where(is_pos, vpos, vneg)
            bucket = jnp.where(
                is_pos, (lo_pos + 15) - _iota16(), lo_neg + _iota16()
            )
            cum = cnt_before + plsc.cumsum(v)
            prev = cum - v
            first = (cum >= need) & (prev < need)
            bits_pos = bucket << 5 << 15
            bits_neg = (((bucket + 1) << 5) << 15) - 1
            ebits = jnp.where(bucket < 2048, bits_pos, bits_neg)
            ecand = lax.bitcast_convert_type(ebits, _F32)
            esel = jnp.sum(jnp.where(first, ecand, jnp.float32(0.0)))
            total = jnp.sum(v)
            return cnt_before + total, efound + esel

        _, ethr = lax.fori_loop(0, 256, scan_loop, (jnp.int32(0), jnp.float32(0.0)))
        ebuf[...] = jnp.full((16,), ethr, _F32)
        pltpu.sync_copy(ebuf, ethr_hbm.at[pl.ds(b * 16, 16)])


def _k3_body(flat_hbm, ethr_hbm, cval_hbm, cidx_hbm, buf, cv, ci, ebuf):
    wid = _wid()
    base_col = wid * SLICE

    def batch_loop(b, _):
        pltpu.sync_copy(ethr_hbm.at[pl.ds(b * 16, 16)], ebuf)
        e = ebuf[...]

        def init_loop(j, _):
            cv[pl.ds(j * 16, 16)] = jnp.full((16,), -jnp.inf, _F32)
            ci[pl.ds(j * 16, 16)] = jnp.zeros((16,), _I32)
            return 0

        lax.fori_loop(0, CAPW // 16, init_loop, 0)

        def chunk_loop(c, cnt):
            pltpu.sync_copy(
                flat_hbm.at[pl.ds(b * FLAT + base_col + c * CHUNK, CHUNK)], buf
            )

            def vec_loop(i, cnt):
                v = buf[pl.ds(i * 16, 16)]
                m = v >= e

                @pl.when(jnp.any(m))
                def _():
                    sel = jnp.where(m, 1, 0).astype(_I32)
                    pos = cnt + plsc.cumsum(sel) - 1
                    okm = m & (pos < CAPW)
                    plsc.store_scatter(cv, [pos], v, mask=okm)
                    idxv = base_col + c * CHUNK + i * 16 + _iota16()
                    plsc.store_scatter(ci, [pos], idxv, mask=okm)

                cnt = cnt + plsc.all_reduce_population_count(m)
                return cnt

            return lax.fori_loop(0, CHUNK // 16, vec_loop, cnt)

        lax.fori_loop(0, NCHUNK, chunk_loop, jnp.zeros((16,), _I32))
        pltpu.sync_copy(cv, cval_hbm.at[pl.ds(b * CAP + wid * CAPW, CAPW)])
        pltpu.sync_copy(ci, cidx_hbm.at[pl.ds(b * CAP + wid * CAPW, CAPW)])
        return 0

    lax.fori_loop(0, B, batch_loop, 0)


def _refine_scan(hist_ref, need, descending):
    def scan_loop(j, carry):
        cnt_before, tfound, afound = carry
        if descending:
            lo = 240 - 16 * j
            v = lax.rev(hist_ref[pl.ds(lo, 16)], (0,))
            bucket = (lo + 15) - _iota16()
        else:
            lo = 16 * j
            v = hist_ref[pl.ds(lo, 16)]
            bucket = lo + _iota16()
        cum = cnt_before + plsc.cumsum(v)
        prev = cum - v
        first = (cum >= need) & (prev < need)
        tsel = jnp.sum(jnp.where(first, bucket, 0))
        asel = jnp.sum(jnp.where(first, prev, 0))
        return cnt_before + jnp.sum(v), tfound + tsel, afound + asel

    _, t, a = lax.fori_loop(
        0, 16, scan_loop, (jnp.int32(0), jnp.int32(0), jnp.int32(0))
    )
    return t, a


def _k4_body(
    cval_hbm, cidx_hbm, boxes_hbm, scale0_hbm, scale1_hbm,
    scores_hbm, labels_hbm, boxesout_hbm,
    val8k, idx8k, smk8k, hist256, wsm, wix, ord_smk, ord_idx,
    ebox, comp, scbuf, lbbuf, bxbuf, s0buf, s1buf, sem,
):
    wid = _wid()
    nvec = CAP // 16

    @pl.when(wid < B)
    def _():
        b = wid
        pltpu.sync_copy(cval_hbm.at[pl.ds(b * CAP, CAP)], val8k)
        pltpu.sync_copy(cidx_hbm.at[pl.ds(b * CAP, CAP)], idx8k)
        pltpu.sync_copy(scale0_hbm.at[pl.ds(b * 16, 16)], s0buf)
        pltpu.sync_copy(scale1_hbm.at[pl.ds(b * 16, 16)], s1buf)

        def smk_loop(i, _):
            v = val8k[pl.ds(i * 16, 16)]
            smk8k[pl.ds(i * 16, 16)] = _smk(lax.bitcast_convert_type(v, _I32))
            return 0

        lax.fori_loop(0, nvec, smk_loop, 0)

        def zero_hist(j, _):
            hist256[pl.ds(j * 16, 16)] = jnp.zeros((16,), _I32)
            return 0

        prefix = jnp.int32(0)
        need = jnp.int32(TOPK)
        for level in range(4):
            shift = 24 - 8 * level
            lax.fori_loop(0, 16, zero_hist, 0)

            def hist_loop(i, _, shift=shift, level=level, prefix=prefix):
                m = smk8k[pl.ds(i * 16, 16)]
                if level == 0:
                    bucket = (lax.shift_right_arithmetic(m, 24) & 255) ^ 128
                    plsc.addupdate_scatter(hist256, [bucket], _ones16())
                else:
                    hi = jnp.full((16,), prefix >> (shift + 8), _I32)
                    valid = lax.shift_right_arithmetic(m, shift + 8) == hi
                    bucket = lax.shift_right_arithmetic(m, shift) & 255
                    plsc.addupdate_scatter(
                        hist256, [bucket], _ones16(), mask=valid
                    )
                return 0

            lax.fori_loop(0, nvec, hist_loop, 0)
            t, above = _refine_scan(hist256, need, descending=True)
            if level == 0:
                prefix = (t ^ 128) << 24
            else:
                prefix = prefix | (t << shift)
            need = need - above

        kkey = prefix

        kvec = jnp.full((16,), kkey, _I32)
        prefix2 = jnp.int32(0)
        for level in range(3):
            shift = 16 - 8 * level
            lax.fori_loop(0, 16, zero_hist, 0)

            def hist_loop2(i, _, shift=shift, level=level, prefix2=prefix2):
                m = smk8k[pl.ds(i * 16, 16)]
                x = idx8k[pl.ds(i * 16, 16)]
                valid = m == kvec
                if level > 0:
                    hi = jnp.full((16,), prefix2 >> (shift + 8), _I32)
                    valid = valid & (lax.shift_right_arithmetic(x, shift + 8) == hi)
                bucket = lax.shift_right_arithmetic(x, shift) & 255
                plsc.addupdate_scatter(hist256, [bucket], _ones16(), mask=valid)
                return 0

            lax.fori_loop(0, nvec, hist_loop2, 0)
            t2, above2 = _refine_scan(hist256, need, descending=False)
            prefix2 = prefix2 | (t2 << shift)
            need = need - above2

        jthr = jnp.full((16,), prefix2, _I32)

        def init_w(j, _):
            wsm[pl.ds(j * 16, 16)] = jnp.full((16,), jnp.int32(-2147483648), _I32)
            wix[pl.ds(j * 16, 16)] = jnp.full((16,), jnp.int32(2147483647), _I32)
            return 0

        lax.fori_loop(0, WPAD // 16, init_w, 0)

        def compact_loop(i, cnt):
            m = smk8k[pl.ds(i * 16, 16)]
            x = idx8k[pl.ds(i * 16, 16)]
            win = (m > kvec) | ((m == kvec) & (x <= jthr))

            @pl.when(jnp.any(win))
            def _():
                sel = jnp.where(win, 1, 0).astype(_I32)
                pos = cnt + plsc.cumsum(sel) - 1
                okm = win & (pos < WPAD)
                plsc.store_scatter(wsm, [pos], m, mask=okm)
                plsc.store_scatter(wix, [pos], x, mask=okm)

            return cnt + plsc.all_reduce_population_count(win)

        lax.fori_loop(0, nvec, compact_loop, jnp.zeros((16,), _I32))

        def init_ord(j, _):
            ord_idx[pl.ds(j * 16, 16)] = jnp.zeros((16,), _I32)
            ord_smk[pl.ds(j * 16, 16)] = jnp.zeros((16,), _I32)
            return 0

        lax.fori_loop(0, ORD // 16, init_ord, 0)

        def rank_outer(vi, _):
            si = wsm[pl.ds(vi * 16, 16)]
            xi = wix[pl.ds(vi * 16, 16)]

            def rank_inner(j, r):
                jj = jnp.full((16,), j, _I32)
                sj = plsc.load_gather(wsm, [jj])
                xj = plsc.load_gather(wix, [jj])
                gt = (sj > si) | ((sj == si) & (xj < xi))
                return r + jnp.where(gt, 1, 0).astype(_I32)

            r = lax.fori_loop(0, WPAD, rank_inner, jnp.zeros((16,), _I32))
            okm = r < TOPK
            plsc.store_scatter(ord_smk, [r], si, mask=okm)
            plsc.store_scatter(ord_idx, [r], xi, mask=okm)
            return 0

        lax.fori_loop(0, WPAD // 16, rank_outer, 0)

        def out_loop(j, _):
            sm = ord_smk[pl.ds(j * 16, 16)]
            v = lax.bitcast_convert_type(_smk(sm), _F32)
            s = jnp.float32(1.0) / (jnp.float32(1.0) + jnp.exp(-v))
            scbuf[pl.ds(j * 16, 16)] = s
            x = ord_idx[pl.ds(j * 16, 16)]
            lbbuf[pl.ds(j * 16, 16)] = x % NC_CLS
            return 0

        lax.fori_loop(0, OUTW // 16, out_loop, 0)

        ebase = b * (NQ * 4)
        for j in range(ORD // 16):
            x = ord_idx[pl.ds(j * 16, 16)]
            q4 = (x // NC_CLS) * 4 + ebase
            for c in range(4):
                flatpos = c * ORD + j * 16
                ebox[flatpos // 128, pl.ds(flatpos % 128, 16)] = q4 + c

        for c in range(4):
            for r in range(3):
                pltpu.async_copy(
                    boxes_hbm.at[ebox.at[c * 3 + r]],
                    comp.at[pl.ds(c * ORD + r * 128, 128)],
                    sem,
                ).wait()

        sx = s0buf[...]
        sy = s1buf[...]
        for j in range(OUTW // 16):
            cx = comp[pl.ds(0 * ORD + j * 16, 16)]
            cy = comp[pl.ds(1 * ORD + j * 16, 16)]
            w = comp[pl.ds(2 * ORD + j * 16, 16)]
            h = comp[pl.ds(3 * ORD + j * 16, 16)]
            half = jnp.float32(0.5)
            x1 = (cx - half * w) * sx
            y1 = (cy - half * h) * sy
            x2 = (cx + half * w) * sx
            y2 = (cy + half * h) * sy
            base4 = (j * 16 + _iota16()) * 4
            plsc.store_scatter(bxbuf, [base4], x1)
            plsc.store_scatter(bxbuf, [base4 + 1], y1)
            plsc.store_scatter(bxbuf, [base4 + 2], x2)
            plsc.store_scatter(bxbuf, [base4 + 3], y2)

        pltpu.sync_copy(scbuf, scores_hbm.at[pl.ds(b * OUTW, OUTW)])
        pltpu.sync_copy(lbbuf, labels_hbm.at[pl.ds(b * OUTW, OUTW)])
        pltpu.sync_copy(bxbuf, boxesout_hbm.at[pl.ds(b * OUTW * 4, OUTW * 4)])


_k1 = functools.partial(
    pl.kernel,
    _k1_body,
    out_type=jax.ShapeDtypeStruct((NW * B * NB,), _I32),
    mesh=_MESH,
    compiler_params=_CPARAMS,
    scratch_types=[
        pltpu.VMEM((CHUNK,), _F32),
        pltpu.VMEM((NB,), _I32),
    ],
)

_k2 = functools.partial(
    pl.kernel,
    _k2_body,
    out_type=jax.ShapeDtypeStruct((B * 16,), _F32),
    mesh=_MESH,
    compiler_params=_CPARAMS,
    scratch_types=[
        pltpu.VMEM((NB,), _I32),
        pltpu.VMEM((NB,), _I32),
        pltpu.VMEM((16,), _F32),
    ],
)

_k3 = functools.partial(
    pl.kernel,
    _k3_body,
    out_type=(
        jax.ShapeDtypeStruct((B * CAP,), _F32),
        jax.ShapeDtypeStruct((B * CAP,), _I32),
    ),
    mesh=_MESH,
    compiler_params=_CPARAMS,
    scratch_types=[
        pltpu.VMEM((CHUNK,), _F32),
        pltpu.VMEM((CAPW,), _F32),
        pltpu.VMEM((CAPW,), _I32),
        pltpu.VMEM((16,), _F32),
    ],
)

_k4 = functools.partial(
    pl.kernel,
    _k4_body,
    out_type=(
        jax.ShapeDtypeStruct((B * OUTW,), _F32),
        jax.ShapeDtypeStruct((B * OUTW,), _I32),
        jax.ShapeDtypeStruct((B * OUTW * 4,), _F32),
    ),
    mesh=_MESH,
    compiler_params=_CPARAMS,
    scratch_types=[
        pltpu.VMEM((CAP,), _F32),
        pltpu.VMEM((CAP,), _I32),
        pltpu.VMEM((CAP,), _I32),
        pltpu.VMEM((256,), _I32),
        pltpu.VMEM((WPAD,), _I32),
        pltpu.VMEM((WPAD,), _I32),
        pltpu.VMEM((ORD,), _I32),
        pltpu.VMEM((ORD,), _I32),
        pltpu.VMEM((12, 128), _I32),
        pltpu.VMEM((4 * ORD,), _F32),
        pltpu.VMEM((OUTW,), _F32),
        pltpu.VMEM((OUTW,), _I32),
        pltpu.VMEM((OUTW * 4,), _F32),
        pltpu.VMEM((16,), _F32),
        pltpu.VMEM((16,), _F32),
        pltpu.SemaphoreType.DMA,
    ],
)


def kernel(pred_logits, pred_boxes, orig_sizes):
    flat = pred_logits.reshape(B * FLAT)
    boxes_flat = pred_boxes.reshape(B * NQ * 4)
    scale0 = jnp.broadcast_to(orig_sizes[:, 0:1], (B, 16)).reshape(B * 16)
    scale1 = jnp.broadcast_to(orig_sizes[:, 1:2], (B, 16)).reshape(B * 16)

    hist = _k1()(flat)
    ethr = _k2()(hist)
    cval, cidx = _k3()(flat, ethr)
    scores, labels, boxes = _k4()(cval, cidx, boxes_flat, scale0, scale1)

    labels_out = labels.reshape(B, OUTW)[:, :TOPK]
    scores_out = scores.reshape(B, OUTW)[:, :TOPK]
    boxes_out = boxes.reshape(B, OUTW * 4)[:, : TOPK * 4].reshape(B, TOPK, 4)
    return labels_out, boxes_out, scores_out


# speedup vs baseline: 24.5445x; 3.6415x over previous
import functools

import jax
import jax.numpy as jnp
from jax import lax
from jax.experimental import pallas as pl
from jax.experimental.pallas import tpu as pltpu
from jax.experimental.pallas import tpu_sc as plsc

B = 16
NQ = 20000
NC_CLS = 80
FLAT = NQ * NC_CLS
TOPK = 300

NCORE = 2
NSUB = 16
NW = NCORE * NSUB
QW = NQ // 2
QCHUNK = 200
NCHUNK = QW // QCHUNK
NB = 4096
CAPW = 2048
CAP = 2 * CAPW
WPAD = 320
ORD = 384
OUTW = 304

_MESH = plsc.VectorSubcoreMesh(core_axis_name="c", subcore_axis_name="s")
_CPARAMS = pltpu.CompilerParams(needs_layout_passes=False)

_I32 = jnp.int32
_F32 = jnp.float32


def _wid():
    return lax.axis_index("s") * NCORE + lax.axis_index("c")


def _iota16():
    return lax.iota(_I32, 16)


def _ones16():
    return jnp.ones((16,), _I32)


def _smk(bits):
    return bits ^ (lax.shift_right_arithmetic(bits, 31) & jnp.int32(0x7FFFFFFF))


def _stream_pair_loop(logits_hbm, b, p, buf0, buf1, sem0, sem1, process):
    qbase = p * QW

    def src_slice(ch):
        return logits_hbm.at[b, pl.ds(qbase + ch * QCHUNK, QCHUNK), :]

    pltpu.async_copy(src_slice(0), buf0, sem0)

    def pair_loop(t, _):
        c0 = 2 * t
        pltpu.make_async_copy(src_slice(c0), buf0, sem0).wait()
        pltpu.async_copy(src_slice(c0 + 1), buf1, sem1)
        process(buf0, c0)
        pltpu.make_async_copy(src_slice(c0 + 1), buf1, sem1).wait()

        @pl.when(t < NCHUNK // 2 - 1)
        def _():
            pltpu.async_copy(src_slice(c0 + 2), buf0, sem0)

        process(buf1, c0 + 1)
        return 0

    lax.fori_loop(0, NCHUNK // 2, pair_loop, 0)


def _k1_body(logits_hbm, hist_hbm, buf0, buf1, hist, sem0, sem1):
    wid = _wid()
    b = wid // 2
    p = wid % 2

    def zero_loop(j, _):
        hist[pl.ds(j * 16, 16)] = jnp.zeros((16,), _I32)
        return 0

    lax.fori_loop(0, NB // 16, zero_loop, 0)

    def process(buf, ch):
        def q_loop(i, _):
            for qq in range(5):
                for u in range(5):
                    v = buf[i * 5 + qq, pl.ds(u * 16, 16)]
                    bu = lax.bitcast_convert_type(v, jnp.uint32)
                    bucket = (bu >> jnp.uint32(20)).astype(_I32)
                    plsc.addupdate_scatter(hist, [bucket], _ones16())
            return 0

        lax.fori_loop(0, QCHUNK // 5, q_loop, 0)

    _stream_pair_loop(logits_hbm, b, p, buf0, buf1, sem0, sem1, process)
    pltpu.sync_copy(hist, hist_hbm.at[pl.ds(wid * NB, NB)])


def _find_threshold(hist_hbm, b, hbuf, acc):
    def zero_loop(j, _):
        acc[pl.ds(j * 16, 16)] = jnp.zeros((16,), _I32)
        return 0

    lax.fori_loop(0, NB // 16, zero_loop, 0)

    def accum_loop(w, _):
        pltpu.sync_copy(hist_hbm.at[pl.ds((2 * b + w) * NB, NB)], hbuf)

        def add_loop(j, _):
            for u in range(4):
                acc[pl.ds((j * 4 + u) * 16, 16)] += hbuf[pl.ds((j * 4 + u) * 16, 16)]
            return 0

        lax.fori_loop(0, NB // 64, add_loop, 0)
        return 0

    lax.fori_loop(0, 2, accum_loop, 0)

    need = jnp.int32(TOPK)

    def scan_loop(j, carry):
        cnt_before, efound = carry
        is_pos = j < 128
        lo_pos = jnp.maximum(2032 - 16 * j, 0)
        lo_neg = jnp.clip(2048 + 16 * (j - 128), 2048, NB - 16)
        vpos = lax.rev(acc[pl.ds(lo_pos, 16)], (0,))
        vneg = acc[pl.ds(lo_neg, 16)]
        v = jnp.where(is_pos, vpos, vneg)
        bucket = jnp.where(is_pos, (lo_pos + 15) - _iota16(), lo_neg + _iota16())
        cum = cnt_before + plsc.cumsum(v)
        prev = cum - v
        first = (cum >= need) & (prev < need)
        bits_pos = bucket << 5 << 15
        bits_neg = (((bucket + 1) << 5) << 15) - 1
        ebits = jnp.where(bucket < 2048, bits_pos, bits_neg)
        ecand = lax.bitcast_convert_type(ebits, _F32)
        esel = jnp.sum(jnp.where(first, ecand, jnp.float32(0.0)))
        return cnt_before + jnp.sum(v), efound + esel

    _, ethr = lax.fori_loop(0, 256, scan_loop, (jnp.int32(0), jnp.float32(0.0)))
    return jnp.full((16,), ethr, _F32)


def _k3_body(logits_hbm, hist_hbm, cval_hbm, cidx_hbm, buf0, buf1, cv, ci, hbuf, acc, sem0, sem1):
    wid = _wid()
    b = wid // 2
    p = wid % 2

    e = _find_threshold(hist_hbm, b, hbuf, acc)

    def init_loop(j, _):
        cv[pl.ds(j * 16, 16)] = jnp.full((16,), -jnp.inf, _F32)
        ci[pl.ds(j * 16, 16)] = jnp.zeros((16,), _I32)
        return 0

    lax.fori_loop(0, CAPW // 16, init_loop, 0)

    def process(buf, ch):
        idx_base = (p * QW + ch * QCHUNK) * NC_CLS

        def q_loop(i, cnt):
            vs = []
            ms = []
            anym = None
            for qq in range(5):
                for u in range(5):
                    v = buf[i * 5 + qq, pl.ds(u * 16, 16)]
                    m = v >= e
                    vs.append((qq, u, v))
                    ms.append(m)
                    anym = m if anym is None else (anym | m)

            def slow(cnt):
                for k, (qq, u, v) in enumerate(vs):
                    m = ms[k]
                    sel = jnp.where(m, 1, 0).astype(_I32)
                    pos = cnt + plsc.cumsum(sel) - 1
                    okm = m & (pos < CAPW)
                    plsc.store_scatter(cv, [pos], v, mask=okm)
                    idxv = idx_base + (i * 5 + qq) * NC_CLS + u * 16 + _iota16()
                    plsc.store_scatter(ci, [pos], idxv, mask=okm)
                    cnt = cnt + plsc.all_reduce_population_count(m)
                return cnt

            return lax.cond(jnp.any(anym), slow, lambda c_: c_, cnt)

        return q_loop

    qbase = p * QW

    def src_slice(ch):
        return logits_hbm.at[b, pl.ds(qbase + ch * QCHUNK, QCHUNK), :]

    pltpu.async_copy(src_slice(0), buf0, sem0)

    def pair_loop(t, cnt):
        c0 = 2 * t
        pltpu.make_async_copy(src_slice(c0), buf0, sem0).wait()
        pltpu.async_copy(src_slice(c0 + 1), buf1, sem1)
        cnt = lax.fori_loop(0, QCHUNK // 5, process(buf0, c0), cnt)
        pltpu.make_async_copy(src_slice(c0 + 1), buf1, sem1).wait()

        @pl.when(t < NCHUNK // 2 - 1)
        def _():
            pltpu.async_copy(src_slice(c0 + 2), buf0, sem0)

        cnt = lax.fori_loop(0, QCHUNK // 5, process(buf1, c0 + 1), cnt)
        return cnt

    lax.fori_loop(0, NCHUNK // 2, pair_loop, jnp.zeros((16,), _I32))
    pltpu.sync_copy(cv, cval_hbm.at[pl.ds(b * CAP + p * CAPW, CAPW)])
    pltpu.sync_copy(ci, cidx_hbm.at[pl.ds(b * CAP + p * CAPW, CAPW)])


def _refine_scan(hist_ref, need, descending):
    def scan_loop(j, carry):
        cnt_before, tfound, afound = carry
        if descending:
            lo = 240 - 16 * j
            v = lax.rev(hist_ref[pl.ds(lo, 16)], (0,))
            bucket = (lo + 15) - _iota16()
        else:
            lo = 16 * j
            v = hist_ref[pl.ds(lo, 16)]
            bucket = lo + _iota16()
        cum = cnt_before + plsc.cumsum(v)
        prev = cum - v
        first = (cum >= need) & (prev < need)
        tsel = jnp.sum(jnp.where(first, bucket, 0))
        asel = jnp.sum(jnp.where(first, prev, 0))
        return cnt_before + jnp.sum(v), tfound + tsel, afound + asel

    _, t, a = lax.fori_loop(
        0, 16, scan_loop, (jnp.int32(0), jnp.int32(0), jnp.int32(0))
    )
    return t, a


def _k4_body(
    cval_hbm, cidx_hbm, boxes_hbm, scale0_hbm, scale1_hbm,
    scores_hbm, labels_hbm, boxesout_hbm,
    val8k, idx8k, smk8k, hist256, wsm, wix, ord_smk, ord_idx,
    ebox, comp, scbuf, lbbuf, bxbuf, s0buf, s1buf, sem,
):
    wid = _wid()
    nvec = CAP // 16

    @pl.when(wid < B)
    def _():
        b = wid
        pltpu.sync_copy(cval_hbm.at[pl.ds(b * CAP, CAP)], val8k)
        pltpu.sync_copy(cidx_hbm.at[pl.ds(b * CAP, CAP)], idx8k)
        pltpu.sync_copy(scale0_hbm.at[pl.ds(b * 16, 16)], s0buf)
        pltpu.sync_copy(scale1_hbm.at[pl.ds(b * 16, 16)], s1buf)

        def smk_loop(i, _):
            v = val8k[pl.ds(i * 16, 16)]
            smk8k[pl.ds(i * 16, 16)] = _smk(lax.bitcast_convert_type(v, _I32))
            return 0

        lax.fori_loop(0, nvec, smk_loop, 0)

        def zero_hist(j, _):
            hist256[pl.ds(j * 16, 16)] = jnp.zeros((16,), _I32)
            return 0

        prefix = jnp.int32(0)
        need = jnp.int32(TOPK)
        for level in range(4):
            shift = 24 - 8 * level
            lax.fori_loop(0, 16, zero_hist, 0)

            def hist_loop(i, _, shift=shift, level=level, prefix=prefix):
                m = smk8k[pl.ds(i * 16, 16)]
                if level == 0:
                    bucket = (lax.shift_right_arithmetic(m, 24) & 255) ^ 128
                    plsc.addupdate_scatter(hist256, [bucket], _ones16())
                else:
                    hi = jnp.full((16,), prefix >> (shift + 8), _I32)
                    valid = lax.shift_right_arithmetic(m, shift + 8) == hi
                    bucket = lax.shift_right_arithmetic(m, shift) & 255
                    plsc.addupdate_scatter(
                        hist256, [bucket], _ones16(), mask=valid
                    )
                return 0

            lax.fori_loop(0, nvec, hist_loop, 0)
            t, above = _refine_scan(hist256, need, descending=True)
            if level == 0:
                prefix = (t ^ 128) << 24
            else:
                prefix = prefix | (t << shift)
            need = need - above

        kkey = prefix

        kvec = jnp.full((16,), kkey, _I32)
        prefix2 = jnp.int32(0)
        for level in range(3):
            shift = 16 - 8 * level
            lax.fori_loop(0, 16, zero_hist, 0)

            def hist_loop2(i, _, shift=shift, level=level, prefix2=prefix2):
                m = smk8k[pl.ds(i * 16, 16)]
                x = idx8k[pl.ds(i * 16, 16)]
                valid = m == kvec
                if level > 0:
                    hi = jnp.full((16,), prefix2 >> (shift + 8), _I32)
                    valid = valid & (lax.shift_right_arithmetic(x, shift + 8) == hi)
                bucket = lax.shift_right_arithmetic(x, shift) & 255
                plsc.addupdate_scatter(hist256, [bucket], _ones16(), mask=valid)
                return 0

            lax.fori_loop(0, nvec, hist_loop2, 0)
            t2, above2 = _refine_scan(hist256, need, descending=False)
            prefix2 = prefix2 | (t2 << shift)
            need = need - above2

        jthr = jnp.full((16,), prefix2, _I32)

        def init_w(j, _):
            wsm[pl.ds(j * 16, 16)] = jnp.full((16,), jnp.int32(-2147483648), _I32)
            wix[pl.ds(j * 16, 16)] = jnp.full((16,), jnp.int32(2147483647), _I32)
            return 0

        lax.fori_loop(0, WPAD // 16, init_w, 0)

        def compact_loop(i, cnt):
            m = smk8k[pl.ds(i * 16, 16)]
            x = idx8k[pl.ds(i * 16, 16)]
            win = (m > kvec) | ((m == kvec) & (x <= jthr))

            @pl.when(jnp.any(win))
            def _():
                sel = jnp.where(win, 1, 0).astype(_I32)
                pos = cnt + plsc.cumsum(sel) - 1
                okm = win & (pos < WPAD)
                plsc.store_scatter(wsm, [pos], m, mask=okm)
                plsc.store_scatter(wix, [pos], x, mask=okm)

            return cnt + plsc.all_reduce_population_count(win)

        lax.fori_loop(0, nvec, compact_loop, jnp.zeros((16,), _I32))

        def init_ord(j, _):
            ord_idx[pl.ds(j * 16, 16)] = jnp.zeros((16,), _I32)
            ord_smk[pl.ds(j * 16, 16)] = jnp.zeros((16,), _I32)
            return 0

        lax.fori_loop(0, ORD // 16, init_ord, 0)

        def rank_outer(vi, _):
            si = wsm[pl.ds(vi * 16, 16)]
            xi = wix[pl.ds(vi * 16, 16)]

            def rank_inner(j, r):
                for u in range(4):
                    jj = jnp.full((16,), j * 4 + u, _I32)
                    sj = plsc.load_gather(wsm, [jj])
                    xj = plsc.load_gather(wix, [jj])
                    gt = (sj > si) | ((sj == si) & (xj < xi))
                    r = r + jnp.where(gt, 1, 0).astype(_I32)
                return r

            r = lax.fori_loop(0, WPAD // 4, rank_inner, jnp.zeros((16,), _I32))
            okm = r < TOPK
            plsc.store_scatter(ord_smk, [r], si, mask=okm)
            plsc.store_scatter(ord_idx, [r], xi, mask=okm)
            return 0

        lax.fori_loop(0, WPAD // 16, rank_outer, 0)

        def out_loop(j, _):
            sm = ord_smk[pl.ds(j * 16, 16)]
            v = lax.bitcast_convert_type(_smk(sm), _F32)
            s = jnp.float32(1.0) / (jnp.float32(1.0) + jnp.exp(-v))
            scbuf[pl.ds(j * 16, 16)] = s
            x = ord_idx[pl.ds(j * 16, 16)]
            lbbuf[pl.ds(j * 16, 16)] = x % NC_CLS
            return 0

        lax.fori_loop(0, OUTW // 16, out_loop, 0)

        ebase = b * (NQ * 4)
        for j in range(ORD // 16):
            x = ord_idx[pl.ds(j * 16, 16)]
            q4 = (x // NC_CLS) * 4 + ebase
            for c in range(4):
                flatpos = c * ORD + j * 16
                ebox[flatpos // 128, pl.ds(flatpos % 128, 16)] = q4 + c

        for c in range(4):
            for r in range(3):
                pltpu.async_copy(
                    boxes_hbm.at[ebox.at[c * 3 + r]],
                    comp.at[pl.ds(c * ORD + r * 128, 128)],
                    sem,
                ).wait()

        sx = s0buf[...]
        sy = s1buf[...]
        for j in range(OUTW // 16):
            cx = comp[pl.ds(0 * ORD + j * 16, 16)]
            cy = comp[pl.ds(1 * ORD + j * 16, 16)]
            w = comp[pl.ds(2 * ORD + j * 16, 16)]
            h = comp[pl.ds(3 * ORD + j * 16, 16)]
            half = jnp.float32(0.5)
            x1 = (cx - half * w) * sx
            y1 = (cy - half * h) * sy
            x2 = (cx + half * w) * sx
            y2 = (cy + half * h) * sy
            base4 = (j * 16 + _iota16()) * 4
            plsc.store_scatter(bxbuf, [base4], x1)
            plsc.store_scatter(bxbuf, [base4 + 1], y1)
            plsc.store_scatter(bxbuf, [base4 + 2], x2)
            plsc.store_scatter(bxbuf, [base4 + 3], y2)

        pltpu.sync_copy(scbuf, scores_hbm.at[pl.ds(b * OUTW, OUTW)])
        pltpu.sync_copy(lbbuf, labels_hbm.at[pl.ds(b * OUTW, OUTW)])
        pltpu.sync_copy(bxbuf, boxesout_hbm.at[pl.ds(b * OUTW * 4, OUTW * 4)])


_k1 = functools.partial(
    pl.kernel,
    _k1_body,
    out_type=jax.ShapeDtypeStruct((NW * NB,), _I32),
    mesh=_MESH,
    compiler_params=_CPARAMS,
    scratch_types=[
        pltpu.VMEM((QCHUNK, NC_CLS), _F32),
        pltpu.VMEM((QCHUNK, NC_CLS), _F32),
        pltpu.VMEM((NB,), _I32),
        pltpu.SemaphoreType.DMA,
        pltpu.SemaphoreType.DMA,
    ],
)

_k3 = functools.partial(
    pl.kernel,
    _k3_body,
    out_type=(
        jax.ShapeDtypeStruct((B * CAP,), _F32),
        jax.ShapeDtypeStruct((B * CAP,), _I32),
    ),
    mesh=_MESH,
    compiler_params=_CPARAMS,
    scratch_types=[
        pltpu.VMEM((QCHUNK, NC_CLS), _F32),
        pltpu.VMEM((QCHUNK, NC_CLS), _F32),
        pltpu.VMEM((CAPW,), _F32),
        pltpu.VMEM((CAPW,), _I32),
        pltpu.VMEM((NB,), _I32),
        pltpu.VMEM((NB,), _I32),
        pltpu.SemaphoreType.DMA,
        pltpu.SemaphoreType.DMA,
    ],
)

_k4 = functools.partial(
    pl.kernel,
    _k4_body,
    out_type=(
        jax.ShapeDtypeStruct((B * OUTW,), _F32),
        jax.ShapeDtypeStruct((B * OUTW,), _I32),
        jax.ShapeDtypeStruct((B * OUTW * 4,), _F32),
    ),
    mesh=_MESH,
    compiler_params=_CPARAMS,
    scratch_types=[
        pltpu.VMEM((CAP,), _F32),
        pltpu.VMEM((CAP,), _I32),
        pltpu.VMEM((CAP,), _I32),
        pltpu.VMEM((256,), _I32),
        pltpu.VMEM((WPAD,), _I32),
        pltpu.VMEM((WPAD,), _I32),
        pltpu.VMEM((ORD,), _I32),
        pltpu.VMEM((ORD,), _I32),
        pltpu.VMEM((12, 128), _I32),
        pltpu.VMEM((4 * ORD,), _F32),
        pltpu.VMEM((OUTW,), _F32),
        pltpu.VMEM((OUTW,), _I32),
        pltpu.VMEM((OUTW * 4,), _F32),
        pltpu.VMEM((16,), _F32),
        pltpu.VMEM((16,), _F32),
        pltpu.SemaphoreType.DMA,
    ],
)


def kernel(pred_logits, pred_boxes, orig_sizes):
    flat = pred_logits
    boxes_flat = pred_boxes.reshape(B * NQ * 4)
    scale0 = jnp.broadcast_to(orig_sizes[:, 0:1], (B, 16)).reshape(B * 16)
    scale1 = jnp.broadcast_to(orig_sizes[:, 1:2], (B, 16)).reshape(B * 16)

    hist = _k1()(flat)
    cval, cidx = _k3()(flat, hist)
    scores, labels, boxes = _k4()(cval, cidx, boxes_flat, scale0, scale1)

    labels_out = labels.reshape(B, OUTW)[:, :TOPK]
    scores_out = scores.reshape(B, OUTW)[:, :TOPK]
    boxes_out = boxes.reshape(B, OUTW * 4)[:, : TOPK * 4].reshape(B, TOPK, 4)
    return labels_out, boxes_out, scores_out
